# Initial kernel scaffold; baseline (speedup 1.0000x reference)
import jax
import jax.numpy as jnp
from jax.experimental import pallas as pl


def kernel(compressed_matrix):
    raise NotImplementedError("write your pallas kernel here")


# speedup vs baseline: 301.5945x; 301.5945x over previous
import jax
import jax.numpy as jnp
from jax import lax
from jax.experimental import pallas as pl
from jax.experimental.pallas import tpu as pltpu, tpu_sc as plsc

N = 8192
L = N * (N - 1) // 2
NC = 2
NS = 16
NW = NC * NS
ROWS_PER_W = N // NW
NQ = N // 16
STAGE = 16
A8MAX = L - N - 8


def _body(comp, out, rowbuf):
    wid = lax.axis_index("s") * NC + lax.axis_index("c")
    iota = lax.iota(jnp.int32, 16)
    zeros16 = jnp.zeros((16,), jnp.float32)

    def row_body(t, carry):
        i = t * NW + wid
        off = i * (N - 1) - (i * (i - 1)) // 2
        a = off - i - 1
        a8 = jnp.clip((a // 8) * 8, 0, A8MAX)
        r = a - a8
        pltpu.sync_copy(
            comp.at[pl.ds(pl.multiple_of(a8, 8), N + 8)],
            rowbuf.at[pl.ds(STAGE, N + 8)],
        )
        q0 = i >> 4

        def shift_chunk(q, c):
            rowbuf[pl.ds(q * 16, 16)] = rowbuf[pl.ds(STAGE + r + q * 16, 16)]
            return c

        lax.fori_loop(q0, NQ, shift_chunk, 0)

        def zero_chunk(q, c):
            rowbuf[pl.ds(q * 16, 16)] = zeros16
            return c

        lax.fori_loop(0, q0, zero_chunk, 0)
        v = rowbuf[pl.ds(q0 * 16, 16)]
        col = q0 * 16 + iota
        rowbuf[pl.ds(q0 * 16, 16)] = jnp.where(col <= i, 0.0, v)
        pltpu.sync_copy(
            rowbuf.at[pl.ds(0, N)],
            out.at[pl.ds(pl.multiple_of(i * N, 8), N)],
        )
        return carry

    lax.fori_loop(0, ROWS_PER_W, row_body, 0)


def kernel(compressed_matrix):
    mesh = plsc.VectorSubcoreMesh(
        core_axis_name="c", subcore_axis_name="s", num_cores=NC, num_subcores=NS
    )
    f = pl.kernel(
        _body,
        out_type=jax.ShapeDtypeStruct((N * N,), jnp.float32),
        mesh=mesh,
        scratch_types=[
            pltpu.VMEM((STAGE + N + 8, ), jnp.float32),
        ],
    )
    return f(compressed_matrix).reshape(N, N)


# speedup vs baseline: 711.1730x; 2.3580x over previous
import jax
import jax.numpy as jnp
from jax import lax
from jax.experimental import pallas as pl
from jax.experimental.pallas import tpu as pltpu, tpu_sc as plsc

N = 8192
L = N * (N - 1) // 2
NC = 2
NS = 16
NW = NC * NS
T = N // NW
NQ = N // 16
SPAD = 16
A8MAX = L - N - 8


def _body(comp, out, stage0, stage1, ob0, ob1, isem0, isem1, osem0, osem1):
    wid = lax.axis_index("s") * NC + lax.axis_index("c")
    iota = lax.iota(jnp.int32, 16)
    zeros16 = jnp.zeros((16,), jnp.float32)

    @plsc.parallel_loop(0, NQ, unroll=8)
    def _zero(q):
        ob0[pl.ds(q * 16, 16)] = zeros16
        ob1[pl.ds(q * 16, 16)] = zeros16

    def row_of(t):
        return (T - 1 - t) * NW + wid

    def src_slice(t):
        i = row_of(t)
        a = i * (N - 1) - (i * (i - 1)) // 2 - i - 1
        a8 = jnp.clip((a // 8) * 8, 0, A8MAX)
        return a8, a - a8

    def start_in(t, stg, sem):
        a8, _ = src_slice(t)
        pltpu.async_copy(
            comp.at[pl.ds(pl.multiple_of(a8, 8), N + 8)],
            stg.at[pl.ds(8, N + 8)],
            sem,
        )

    def wait_in(stg, sem):
        pltpu.make_async_copy(
            comp.at[pl.ds(0, N + 8)], stg.at[pl.ds(8, N + 8)], sem
        ).wait()

    def start_out(t, ob, sem):
        i = row_of(t)
        pltpu.async_copy(
            ob.at[pl.ds(0, N)],
            out.at[pl.ds(pl.multiple_of(i * N, 8), N)],
            sem,
        )

    def wait_out(ob, sem):
        pltpu.make_async_copy(
            ob.at[pl.ds(0, N)], out.at[pl.ds(0, N)], sem
        ).wait()

    def shift_mask(t, stg, ob):
        i = row_of(t)
        _, r = src_slice(t)
        q0 = i >> 4
        qs = q0 & ~7

        @plsc.parallel_loop(qs, NQ, unroll=8)
        def _sh(q):
            v = stg[pl.ds(8 + r + q * 16, 16)]
            col = q * 16 + iota
            ob[pl.ds(q * 16, 16)] = jnp.where(col <= i, 0.0, v)

    start_in(0, stage0, isem0)

    def pair_body(p, carry):
        t0 = 2 * p
        start_in(t0 + 1, stage1, isem1)
        wait_in(stage0, isem0)

        @pl.when(p > 0)
        def _():
            wait_out(ob0, osem0)

        shift_mask(t0, stage0, ob0)
        start_out(t0, ob0, osem0)

        @pl.when(p < T // 2 - 1)
        def _():
            start_in(t0 + 2, stage0, isem0)

        wait_in(stage1, isem1)

        @pl.when(p > 0)
        def _():
            wait_out(ob1, osem1)

        shift_mask(t0 + 1, stage1, ob1)
        start_out(t0 + 1, ob1, osem1)
        return carry

    lax.fori_loop(0, T // 2, pair_body, 0)
    wait_out(ob0, osem0)
    wait_out(ob1, osem1)


def kernel(compressed_matrix):
    mesh = plsc.VectorSubcoreMesh(
        core_axis_name="c", subcore_axis_name="s", num_cores=NC, num_subcores=NS
    )
    f = pl.kernel(
        _body,
        out_type=jax.ShapeDtypeStruct((N * N,), jnp.float32),
        mesh=mesh,
        scratch_types=[
            pltpu.VMEM((N + SPAD + 8,), jnp.float32),
            pltpu.VMEM((N + SPAD + 8,), jnp.float32),
            pltpu.VMEM((N,), jnp.float32),
            pltpu.VMEM((N,), jnp.float32),
            pltpu.SemaphoreType.DMA,
            pltpu.SemaphoreType.DMA,
            pltpu.SemaphoreType.DMA,
            pltpu.SemaphoreType.DMA,
        ],
    )
    return f(compressed_matrix).reshape(N, N)


# speedup vs baseline: 1897.0533x; 2.6675x over previous
import jax
import jax.numpy as jnp
from jax import lax
from jax.experimental import pallas as pl
from jax.experimental.pallas import tpu as pltpu, tpu_sc as plsc

N = 8192
L = N * (N - 1) // 2
NC = 2
NS = 16
NW = NC * NS
G = 8
C = 2048
KT = N // C
NT = (N // G) * KT
MT = NT // NW
NQ = C // 16
SROW = C + 32
A8MAX = L - C - 16


def _body(comp, out, stage0, stage1, ob0, ob1, zb, isem0, isem1, osem0, osem1):
    wid = lax.axis_index("s") * NC + lax.axis_index("c")
    iota = lax.iota(jnp.int32, 16)
    zeros16 = jnp.zeros((16,), jnp.float32)

    for r in range(G):
        @plsc.parallel_loop(0, NQ, unroll=8)
        def _z(q, _r=r):
            zb[_r, pl.ds(q * 16, 16)] = zeros16

    def tile_of(m):
        idx = m * NW + wid
        g = idx >> 2
        k = idx & 3
        return g, k

    def is_zero_tile(g, k):
        return (k + 1) * C <= G * g

    def row_src(g, k, r):
        i = G * g + r
        b = i * (N - 1) - (i * (i - 1)) // 2 - i - 1 + k * C
        a8 = jnp.clip((b // 8) * 8, 0, A8MAX)
        return i, a8, b - a8

    def issue_in(m, stg, sem):
        g, k = tile_of(m)

        @pl.when(jnp.logical_not(is_zero_tile(g, k)))
        def _():
            for r in range(G):
                _, a8, _2 = row_src(g, k, r)
                pltpu.async_copy(
                    comp.at[pl.ds(pl.multiple_of(a8, 8), C + 16)],
                    stg.at[pl.ds(r * SROW + 8, C + 16)],
                    sem,
                )

    def wait_in(stg, sem):
        for r in range(G):
            pltpu.make_async_copy(
                comp.at[pl.ds(0, C + 16)],
                stg.at[pl.ds(r * SROW + 8, C + 16)],
                sem,
            ).wait()

    def wait_out(ob, osem):
        pltpu.make_async_copy(ob, out.at[0, :, pl.ds(0, C)], osem).wait()

    def process(m, stg, ob, isem, osem, not_first, not_last):
        g, k = tile_of(m)
        zero = is_zero_tile(g, k)
        data = jnp.logical_not(zero)
        full = k * C >= G * g + G

        @pl.when(data)
        def _():
            wait_in(stg, isem)

        @pl.when(not_first)
        def _():
            wait_out(ob, osem)

        @pl.when(full)
        def _():
            for r in range(G):
                _, _2, rr = row_src(g, k, r)

                @plsc.parallel_loop(0, NQ, unroll=8)
                def _sh(q, _r=r, _rr=rr):
                    ob[_r, pl.ds(q * 16, 16)] = stg[
                        pl.ds(_r * SROW + 8 + _rr + q * 16, 16)
                    ]

        @pl.when(jnp.logical_and(data, jnp.logical_not(full)))
        def _():
            for r in range(G):
                i, _2, rr = row_src(g, k, r)
                colbase = k * C + iota

                @plsc.parallel_loop(0, NQ, unroll=8)
                def _shm(q, _r=r, _rr=rr, _i=i, _cb=colbase):
                    v = stg[pl.ds(_r * SROW + 8 + _rr + q * 16, 16)]
                    col = _cb + q * 16
                    ob[_r, pl.ds(q * 16, 16)] = jnp.where(col <= _i, 0.0, v)

        dst = out.at[g, :, pl.ds(pl.multiple_of(k * C, 128), C)]

        @pl.when(data)
        def _():
            pltpu.async_copy(ob, dst, osem)

        @pl.when(zero)
        def _():
            pltpu.async_copy(zb, dst, osem)

        @pl.when(not_last)
        def _():
            issue_in(m + 2, stg, isem)

    issue_in(0, stage0, isem0)
    issue_in(1, stage1, isem1)

    def pair_body(pp, carry):
        m0 = 2 * pp
        process(m0, stage0, ob0, isem0, osem0, pp > 0, pp < MT // 2 - 1)
        process(m0 + 1, stage1, ob1, isem1, osem1, pp > 0, pp < MT // 2 - 1)
        return carry

    lax.fori_loop(0, MT // 2, pair_body, 0)
    wait_out(ob0, osem0)
    wait_out(ob1, osem1)


def kernel(compressed_matrix):
    mesh = plsc.VectorSubcoreMesh(
        core_axis_name="c", subcore_axis_name="s", num_cores=NC, num_subcores=NS
    )
    f = pl.kernel(
        _body,
        out_type=jax.ShapeDtypeStruct((N // G, G, N), jnp.float32),
        mesh=mesh,
        scratch_types=[
            pltpu.VMEM((G * SROW,), jnp.float32),
            pltpu.VMEM((G * SROW,), jnp.float32),
            pltpu.VMEM((G, C), jnp.float32),
            pltpu.VMEM((G, C), jnp.float32),
            pltpu.VMEM((G, C), jnp.float32),
            pltpu.SemaphoreType.DMA,
            pltpu.SemaphoreType.DMA,
            pltpu.SemaphoreType.DMA,
            pltpu.SemaphoreType.DMA,
        ],
    )
    return f(compressed_matrix).reshape(N, N)


# speedup vs baseline: 2040.6348x; 1.0757x over previous
import jax
import jax.numpy as jnp
from jax import lax
from jax.experimental import pallas as pl
from jax.experimental.pallas import tpu as pltpu, tpu_sc as plsc

N = 8192
L = N * (N - 1) // 2
NC = 2
NS = 16
NW = NC * NS
G = 8
C = 2048
KT = N // C
NT = (N // G) * KT
MT = NT // NW
NQ = C // 16
SROW = C + 32
A8MAX = L - C - 16


def _body(comp, out, stage0, stage1, ob0, ob1, zb, isem0, isem1, osem0, osem1):
    wid = lax.axis_index("s") * NC + lax.axis_index("c")
    iota = lax.iota(jnp.int32, 16)
    zeros16 = jnp.zeros((16,), jnp.float32)

    for r in range(G):
        @plsc.parallel_loop(0, NQ, unroll=8)
        def _z(q, _r=r):
            zb[_r, pl.ds(q * 16, 16)] = zeros16

    def tile_of(m):
        idx = m * NW + wid
        g = idx & (N // G - 1)
        k = idx >> 10
        return g, k

    def is_zero_tile(g, k):
        return (k + 1) * C <= G * g

    def row_src(g, k, r):
        i = G * g + r
        b = i * (N - 1) - (i * (i - 1)) // 2 - i - 1 + k * C
        a8 = jnp.clip((b // 8) * 8, 0, A8MAX)
        return i, a8, b - a8

    def issue_in(m, stg, sem):
        g, k = tile_of(m)

        @pl.when(jnp.logical_not(is_zero_tile(g, k)))
        def _():
            for r in range(G):
                _, a8, _2 = row_src(g, k, r)
                pltpu.async_copy(
                    comp.at[pl.ds(pl.multiple_of(a8, 8), C + 16)],
                    stg.at[pl.ds(r * SROW + 8, C + 16)],
                    sem,
                )

    def wait_in(m, stg, sem):
        pltpu.make_async_copy(
            comp.at[pl.ds(0, G * (C + 16))],
            stg.at[pl.ds(0, G * (C + 16))],
            sem,
        ).wait()

    def wait_out(ob, osem):
        pltpu.make_async_copy(ob, out.at[0, :, pl.ds(0, C)], osem).wait()

    def process(m, stg, ob, isem, osem, not_first, not_last):
        g, k = tile_of(m)
        zero = is_zero_tile(g, k)
        data = jnp.logical_not(zero)
        full = k * C >= G * g + G

        @pl.when(data)
        def _():
            wait_in(m, stg, isem)

        @pl.when(not_first)
        def _():
            wait_out(ob, osem)

        @pl.when(full)
        def _():
            for r in range(G):
                _, _2, rr = row_src(g, k, r)

                @plsc.parallel_loop(0, NQ, unroll=8)
                def _sh(q, _r=r, _rr=rr):
                    ob[_r, pl.ds(q * 16, 16)] = stg[
                        pl.ds(_r * SROW + 8 + _rr + q * 16, 16)
                    ]

        @pl.when(jnp.logical_and(data, jnp.logical_not(full)))
        def _():
            for r in range(G):
                i, _2, rr = row_src(g, k, r)
                colbase = k * C + iota

                @plsc.parallel_loop(0, NQ, unroll=8)
                def _shm(q, _r=r, _rr=rr, _i=i, _cb=colbase):
                    v = stg[pl.ds(_r * SROW + 8 + _rr + q * 16, 16)]
                    col = _cb + q * 16
                    ob[_r, pl.ds(q * 16, 16)] = jnp.where(col <= _i, 0.0, v)

        dst = out.at[g, :, pl.ds(pl.multiple_of(k * C, 128), C)]

        @pl.when(data)
        def _():
            pltpu.async_copy(ob, dst, osem)

        @pl.when(zero)
        def _():
            pltpu.async_copy(zb, dst, osem)

        @pl.when(not_last)
        def _():
            issue_in(m + 2, stg, isem)

    issue_in(0, stage0, isem0)
    issue_in(1, stage1, isem1)

    def pair_body(pp, carry):
        m0 = 2 * pp
        process(m0, stage0, ob0, isem0, osem0, pp > 0, pp < MT // 2 - 1)
        process(m0 + 1, stage1, ob1, isem1, osem1, pp > 0, pp < MT // 2 - 1)
        return carry

    lax.fori_loop(0, MT // 2, pair_body, 0)
    wait_out(ob0, osem0)
    wait_out(ob1, osem1)


def kernel(compressed_matrix):
    mesh = plsc.VectorSubcoreMesh(
        core_axis_name="c", subcore_axis_name="s", num_cores=NC, num_subcores=NS
    )
    f = pl.kernel(
        _body,
        out_type=jax.ShapeDtypeStruct((N // G, G, N), jnp.float32),
        mesh=mesh,
        scratch_types=[
            pltpu.VMEM((G * SROW,), jnp.float32),
            pltpu.VMEM((G * SROW,), jnp.float32),
            pltpu.VMEM((G, C), jnp.float32),
            pltpu.VMEM((G, C), jnp.float32),
            pltpu.VMEM((G, C), jnp.float32),
            pltpu.SemaphoreType.DMA,
            pltpu.SemaphoreType.DMA,
            pltpu.SemaphoreType.DMA,
            pltpu.SemaphoreType.DMA,
        ],
    )
    return f(compressed_matrix).reshape(N, N)
